# baseline (speedup 1.0000x reference)
import functools

import jax
import jax.numpy as jnp
from jax.experimental import pallas as pl
from jax.experimental.pallas import tpu as pltpu

_GROUP_RANK = 16
_K_GROUPS = 64
_T = 256
_FB = 1024


def _norms(pre_ref, nrm_ref, j, gpb):
    blk = pre_ref[j]
    sq = blk * blk
    hi = sq.astype(jnp.bfloat16)
    lo = (sq - hi.astype(jnp.float32)).astype(jnp.bfloat16)
    g_ind = (jax.lax.broadcasted_iota(jnp.int32, (_FB, gpb), 0)
             // _GROUP_RANK
             == jax.lax.broadcasted_iota(jnp.int32, (_FB, gpb), 1)
             ).astype(jnp.bfloat16)
    nrm_ref[j] = (jnp.dot(hi, g_ind, preferred_element_type=jnp.float32)
                  + jnp.dot(lo, g_ind, preferred_element_type=jnp.float32))


def _prep(pre_ref, msk_ref, mop_ref, j, gpb):
    gt_ind = (jax.lax.broadcasted_iota(jnp.int32, (gpb, _FB), 0)
              == jax.lax.broadcasted_iota(jnp.int32, (gpb, _FB), 1)
              // _GROUP_RANK).astype(jnp.bfloat16)
    mfeat = jnp.dot(msk_ref[j], gt_ind,
                    preferred_element_type=jnp.float32).astype(jnp.bfloat16)
    mop_ref[j % 2] = pre_ref[j].astype(jnp.bfloat16) * mfeat


def _fused(x_ref, we_ref, wd_ref, be_ref, bd_ref, out_ref,
           pre_ref, nrm_ref, msk_ref, mop_ref, *, nfb, gpb):
    p = pl.program_id(1)
    t_tile = pre_ref.shape[1]

    @pl.when(p < nfb)
    def _encode():
        blk = jnp.dot(x_ref[...], we_ref[...],
                      preferred_element_type=jnp.float32)
        pre_ref[p] = blk + be_ref[p]

        @pl.when(p > 0)
        def _n():
            _norms(pre_ref, nrm_ref, p - 1, gpb)

    @pl.when(p == nfb)
    def _select():
        _norms(pre_ref, nrm_ref, nfb - 1, gpb)
        bits = jax.lax.bitcast_convert_type(nrm_ref[...], jnp.int32)

        def body(_, carry):
            lo_b, hi_b = carry
            mid = lo_b + ((hi_b - lo_b) >> 1)
            cnt = jnp.sum((bits >= mid).astype(jnp.int32), axis=(0, 2),
                          keepdims=True)
            ok = cnt >= _K_GROUPS
            return jnp.where(ok, mid, lo_b), jnp.where(ok, hi_b, mid)

        lo0 = jnp.zeros((1, t_tile, 1), jnp.int32)
        hi0 = jnp.full((1, t_tile, 1), jnp.int32(0x7F800000))
        thr, _ = jax.lax.fori_loop(0, 31, body, (lo0, hi0))
        msk_ref[...] = (bits >= thr).astype(jnp.bfloat16)
        _prep(pre_ref, msk_ref, mop_ref, 0, gpb)

    @pl.when(p > nfb)
    def _decode():
        b = p - nfb - 1
        acc = jnp.dot(mop_ref[b % 2], wd_ref[...],
                      preferred_element_type=jnp.float32)

        @pl.when(b < nfb - 1)
        def _pn():
            _prep(pre_ref, msk_ref, mop_ref, b + 1, gpb)

        @pl.when(b == 0)
        def _init():
            out_ref[...] = acc + bd_ref[...]

        @pl.when(b > 0)
        def _acc():
            out_ref[...] += acc


def kernel(x, W_enc, W_dec, b_enc, b_dec):
    tokens, d_model = x.shape
    d_sae = W_enc.shape[1]
    nfb = d_sae // _FB
    gpb = _FB // _GROUP_RANK

    x16 = x.astype(jnp.bfloat16)
    we16 = W_enc.astype(jnp.bfloat16)
    wd16 = W_dec.astype(jnp.bfloat16)
    be3 = b_enc.reshape(nfb, 1, _FB)
    bd2 = b_dec.reshape(1, d_model)

    grid = (tokens // _T, 2 * nfb + 1)
    body = functools.partial(_fused, nfb=nfb, gpb=gpb)
    return pl.pallas_call(
        body,
        grid=grid,
        in_specs=[
            pl.BlockSpec((_T, d_model), lambda t, p: (t, 0)),
            pl.BlockSpec((d_model, _FB),
                         lambda t, p, n=nfb: (0, jnp.minimum(p, n - 1))),
            pl.BlockSpec((_FB, d_model),
                         lambda t, p, n=nfb: (jnp.clip(p - n - 1, 0, n - 1), 0)),
            pl.BlockSpec((nfb, 1, _FB), lambda t, p: (0, 0, 0)),
            pl.BlockSpec((1, d_model), lambda t, p: (0, 0)),
        ],
        out_specs=pl.BlockSpec((_T, d_model), lambda t, p: (t, 0)),
        out_shape=jax.ShapeDtypeStruct((tokens, d_model), jnp.float32),
        scratch_shapes=[
            pltpu.VMEM((nfb, _T, _FB), jnp.float32),
            pltpu.VMEM((nfb, _T, gpb), jnp.float32),
            pltpu.VMEM((nfb, _T, gpb), jnp.bfloat16),
            pltpu.VMEM((2, _T, _FB), jnp.bfloat16),
        ],
        compiler_params=pltpu.CompilerParams(
            dimension_semantics=("arbitrary", "arbitrary"),
            vmem_limit_bytes=110 * 1024 * 1024,
        ),
    )(x16, we16, wd16, be3, bd2)


# speedup vs baseline: 1.0779x; 1.0779x over previous
import functools

import jax
import jax.numpy as jnp
from jax.experimental import pallas as pl
from jax.experimental.pallas import tpu as pltpu

_GROUP_RANK = 16
_K_GROUPS = 64
_T = 256
_FB = 1024


def _norms(pre_ref, nrm_ref, j, gpb):
    blk = pre_ref[j]
    sq = blk * blk
    hi = sq.astype(jnp.bfloat16)
    lo = (sq - hi.astype(jnp.float32)).astype(jnp.bfloat16)
    g_ind = (jax.lax.broadcasted_iota(jnp.int32, (_FB, gpb), 0)
             // _GROUP_RANK
             == jax.lax.broadcasted_iota(jnp.int32, (_FB, gpb), 1)
             ).astype(jnp.bfloat16)
    dn = (((0,), (1,)), ((), ()))
    nt = (jax.lax.dot_general(g_ind, hi, dn,
                              preferred_element_type=jnp.float32)
          + jax.lax.dot_general(g_ind, lo, dn,
                                preferred_element_type=jnp.float32))
    nrm_ref[pl.ds(pl.multiple_of(j * gpb, gpb), gpb), :] = nt


def _prep(pre_ref, msk_ref, mop_ref, j, gpb):
    gt_ind = (jax.lax.broadcasted_iota(jnp.int32, (gpb, _FB), 0)
              == jax.lax.broadcasted_iota(jnp.int32, (gpb, _FB), 1)
              // _GROUP_RANK).astype(jnp.bfloat16)
    mg = msk_ref[pl.ds(pl.multiple_of(j * gpb, gpb), gpb), :]
    dn = (((0,), (0,)), ((), ()))
    mfeat = jax.lax.dot_general(mg, gt_ind, dn,
                                preferred_element_type=jnp.float32
                                ).astype(jnp.bfloat16)
    mop_ref[j % 2] = pre_ref[j].astype(jnp.bfloat16) * mfeat


def _fused(x_ref, we_ref, wd_ref, be_ref, bd_ref, out_ref,
           pre_ref, nrm_ref, msk_ref, mop_ref, *, nfb, gpb):
    p = pl.program_id(1)
    t_tile = pre_ref.shape[1]

    @pl.when(p < nfb)
    def _encode():
        blk = jnp.dot(x_ref[...], we_ref[...],
                      preferred_element_type=jnp.float32)
        pre_ref[p] = blk + be_ref[p]
        _norms(pre_ref, nrm_ref, jnp.maximum(p - 1, 0), gpb)

    @pl.when(p == nfb)
    def _select():
        _norms(pre_ref, nrm_ref, nfb - 1, gpb)
        bits = jax.lax.bitcast_convert_type(nrm_ref[...], jnp.int32)

        def body(_, carry):
            lo_b, hi_b = carry
            mid = lo_b + ((hi_b - lo_b) >> 1)
            cnt = jnp.sum((bits >= mid).astype(jnp.int32), axis=0,
                          keepdims=True)
            ok = cnt >= _K_GROUPS
            return jnp.where(ok, mid, lo_b), jnp.where(ok, hi_b, mid)

        lo0 = jnp.zeros((1, t_tile), jnp.int32)
        hi0 = jnp.full((1, t_tile), jnp.int32(0x7F800000))
        thr, _ = jax.lax.fori_loop(0, 31, body, (lo0, hi0))
        msk_ref[...] = (bits >= thr).astype(jnp.bfloat16)
        _prep(pre_ref, msk_ref, mop_ref, 0, gpb)

    @pl.when(p > nfb)
    def _decode():
        b = p - nfb - 1
        acc = jnp.dot(mop_ref[b % 2], wd_ref[...],
                      preferred_element_type=jnp.float32)
        _prep(pre_ref, msk_ref, mop_ref, jnp.minimum(b + 1, nfb - 1), gpb)

        @pl.when(b == 0)
        def _init():
            out_ref[...] = acc + bd_ref[...]

        @pl.when(b > 0)
        def _acc():
            out_ref[...] += acc


def kernel(x, W_enc, W_dec, b_enc, b_dec):
    tokens, d_model = x.shape
    d_sae = W_enc.shape[1]
    nfb = d_sae // _FB
    gpb = _FB // _GROUP_RANK

    x16 = x.astype(jnp.bfloat16)
    we16 = W_enc.astype(jnp.bfloat16)
    wd16 = W_dec.astype(jnp.bfloat16)
    be3 = b_enc.reshape(nfb, 1, _FB)
    bd2 = b_dec.reshape(1, d_model)

    grid = (tokens // _T, 2 * nfb + 1)
    body = functools.partial(_fused, nfb=nfb, gpb=gpb)
    return pl.pallas_call(
        body,
        grid=grid,
        in_specs=[
            pl.BlockSpec((_T, d_model), lambda t, p: (t, 0)),
            pl.BlockSpec((d_model, _FB),
                         lambda t, p, n=nfb: (0, jnp.minimum(p, n - 1))),
            pl.BlockSpec((_FB, d_model),
                         lambda t, p, n=nfb: (jnp.clip(p - n - 1, 0, n - 1), 0)),
            pl.BlockSpec((nfb, 1, _FB), lambda t, p: (0, 0, 0)),
            pl.BlockSpec((1, d_model), lambda t, p: (0, 0)),
        ],
        out_specs=pl.BlockSpec((_T, d_model), lambda t, p: (t, 0)),
        out_shape=jax.ShapeDtypeStruct((tokens, d_model), jnp.float32),
        scratch_shapes=[
            pltpu.VMEM((nfb, _T, _FB), jnp.float32),
            pltpu.VMEM((nfb * gpb, _T), jnp.float32),
            pltpu.VMEM((nfb * gpb, _T), jnp.bfloat16),
            pltpu.VMEM((2, _T, _FB), jnp.bfloat16),
        ],
        compiler_params=pltpu.CompilerParams(
            dimension_semantics=("arbitrary", "arbitrary"),
            vmem_limit_bytes=64 * 1024 * 1024,
        ),
    )(x16, we16, wd16, be3, bd2)


# speedup vs baseline: 1.1684x; 1.0840x over previous
import functools

import jax
import jax.numpy as jnp
from jax.experimental import pallas as pl
from jax.experimental.pallas import tpu as pltpu

_GROUP_RANK = 16
_K_GROUPS = 64
_T = 256
_FB = 1024


def _norms(pre_ref, nrm_ref, j, gpb):
    blk = pre_ref[j]
    sq = blk * blk
    hi = sq.astype(jnp.bfloat16)
    lo = (sq - hi.astype(jnp.float32)).astype(jnp.bfloat16)
    g_ind = (jax.lax.broadcasted_iota(jnp.int32, (_FB, gpb), 0)
             // _GROUP_RANK
             == jax.lax.broadcasted_iota(jnp.int32, (_FB, gpb), 1)
             ).astype(jnp.bfloat16)
    dn = (((0,), (1,)), ((), ()))
    nt = (jax.lax.dot_general(g_ind, hi, dn,
                              preferred_element_type=jnp.float32)
          + jax.lax.dot_general(g_ind, lo, dn,
                                preferred_element_type=jnp.float32))
    nrm_ref[pl.ds(pl.multiple_of(j * gpb, gpb), gpb), :] = nt


def _prep(pre_ref, msk_ref, mop_ref, j, gpb):
    gt_ind = (jax.lax.broadcasted_iota(jnp.int32, (gpb, _FB), 0)
              == jax.lax.broadcasted_iota(jnp.int32, (gpb, _FB), 1)
              // _GROUP_RANK).astype(jnp.bfloat16)
    mg = msk_ref[pl.ds(pl.multiple_of(j * gpb, gpb), gpb), :]
    dn = (((0,), (0,)), ((), ()))
    mfeat = jax.lax.dot_general(mg, gt_ind, dn,
                                preferred_element_type=jnp.float32
                                ).astype(jnp.bfloat16)
    mop_ref[j % 2] = pre_ref[j].astype(jnp.bfloat16) * mfeat


def _fused(x_ref, we_ref, wd_ref, be_ref, bd_ref, out_ref,
           pre_ref, nrm_ref, msk_ref, mop_ref, *, nfb, gpb):
    p = pl.program_id(1)
    t_tile = pre_ref.shape[1]

    @pl.when(p < nfb)
    def _encode():
        _norms(pre_ref, nrm_ref, jnp.maximum(p - 1, 0), gpb)
        h = t_tile // 2
        for i in range(2):
            blk = jnp.dot(x_ref[i * h:(i + 1) * h, :], we_ref[...],
                          preferred_element_type=jnp.float32)
            pre_ref[p, i * h:(i + 1) * h, :] = blk + be_ref[p]

    @pl.when(p == nfb)
    def _select():
        _norms(pre_ref, nrm_ref, nfb - 1, gpb)
        bits = jax.lax.bitcast_convert_type(nrm_ref[...], jnp.int32)

        def body(_, carry):
            lo_b, hi_b = carry
            mid = lo_b + ((hi_b - lo_b) >> 1)
            cnt = jnp.sum((bits >= mid).astype(jnp.int32), axis=0,
                          keepdims=True)
            ok = cnt >= _K_GROUPS
            return jnp.where(ok, mid, lo_b), jnp.where(ok, hi_b, mid)

        lo0 = jnp.zeros((1, t_tile), jnp.int32)
        hi0 = jnp.full((1, t_tile), jnp.int32(0x7F800000))
        thr, _ = jax.lax.fori_loop(0, 31, body, (lo0, hi0))
        msk_ref[...] = (bits >= thr).astype(jnp.bfloat16)
        _prep(pre_ref, msk_ref, mop_ref, 0, gpb)

    @pl.when(p > nfb)
    def _decode():
        b = p - nfb - 1
        h = t_tile // 2
        for i in range(2):
            sl = slice(i * h, (i + 1) * h)
            acc = jnp.dot(mop_ref[b % 2, sl, :], wd_ref[...],
                          preferred_element_type=jnp.float32)
            base = jnp.where(b == 0, bd_ref[...], out_ref[sl, :])
            out_ref[sl, :] = base + acc
        _prep(pre_ref, msk_ref, mop_ref, jnp.minimum(b + 1, nfb - 1), gpb)


def kernel(x, W_enc, W_dec, b_enc, b_dec):
    tokens, d_model = x.shape
    d_sae = W_enc.shape[1]
    nfb = d_sae // _FB
    gpb = _FB // _GROUP_RANK

    x16 = x.astype(jnp.bfloat16)
    we16 = W_enc.astype(jnp.bfloat16)
    wd16 = W_dec.astype(jnp.bfloat16)
    be3 = b_enc.reshape(nfb, 1, _FB)
    bd2 = b_dec.reshape(1, d_model)

    grid = (tokens // _T, 2 * nfb + 1)
    body = functools.partial(_fused, nfb=nfb, gpb=gpb)
    return pl.pallas_call(
        body,
        grid=grid,
        in_specs=[
            pl.BlockSpec((_T, d_model), lambda t, p: (t, 0)),
            pl.BlockSpec((d_model, _FB),
                         lambda t, p, n=nfb: (0, jnp.minimum(p, n - 1))),
            pl.BlockSpec((_FB, d_model),
                         lambda t, p, n=nfb: (jnp.clip(p - n - 1, 0, n - 1), 0)),
            pl.BlockSpec((nfb, 1, _FB), lambda t, p: (0, 0, 0)),
            pl.BlockSpec((1, d_model), lambda t, p: (0, 0)),
        ],
        out_specs=pl.BlockSpec((_T, d_model), lambda t, p: (t, 0)),
        out_shape=jax.ShapeDtypeStruct((tokens, d_model), jnp.float32),
        scratch_shapes=[
            pltpu.VMEM((nfb, _T, _FB), jnp.float32),
            pltpu.VMEM((nfb * gpb, _T), jnp.float32),
            pltpu.VMEM((nfb * gpb, _T), jnp.bfloat16),
            pltpu.VMEM((2, _T, _FB), jnp.bfloat16),
        ],
        compiler_params=pltpu.CompilerParams(
            dimension_semantics=("arbitrary", "arbitrary"),
            vmem_limit_bytes=64 * 1024 * 1024,
        ),
    )(x16, we16, wd16, be3, bd2)


# speedup vs baseline: 1.3388x; 1.1459x over previous
import functools

import jax
import jax.numpy as jnp
from jax.experimental import pallas as pl
from jax.experimental.pallas import tpu as pltpu

_GROUP_RANK = 16
_K_GROUPS = 64
_T = 256
_FB = 1024


def _norms(pre_ref, nrm_ref, sel, j, gpb):
    blk = pre_ref[sel, j]
    sq = blk * blk
    hi = sq.astype(jnp.bfloat16)
    lo = (sq - hi.astype(jnp.float32)).astype(jnp.bfloat16)
    g_ind = (jax.lax.broadcasted_iota(jnp.int32, (_FB, gpb), 0)
             // _GROUP_RANK
             == jax.lax.broadcasted_iota(jnp.int32, (_FB, gpb), 1)
             ).astype(jnp.bfloat16)
    dn = (((0,), (1,)), ((), ()))
    nt = (jax.lax.dot_general(g_ind, hi, dn,
                              preferred_element_type=jnp.float32)
          + jax.lax.dot_general(g_ind, lo, dn,
                                preferred_element_type=jnp.float32))
    nrm_ref[pl.ds(j * gpb, gpb), :] = nt


def _prep(pre_ref, msk_ref, mop_ref, sel, j, gpb):
    gt_ind = (jax.lax.broadcasted_iota(jnp.int32, (gpb, _FB), 0)
              == jax.lax.broadcasted_iota(jnp.int32, (gpb, _FB), 1)
              // _GROUP_RANK).astype(jnp.bfloat16)
    mg = msk_ref[pl.ds(j * gpb, gpb), :]
    dn = (((0,), (0,)), ((), ()))
    mfeat = jax.lax.dot_general(mg, gt_ind, dn,
                                preferred_element_type=jnp.float32
                                ).astype(jnp.bfloat16)
    mop_ref[jax.lax.rem(j, 2)] = pre_ref[sel, j].astype(jnp.bfloat16) * mfeat


def _fused(x_ref, we_ref, wd_ref, be_ref, bd_ref, out_ref,
           pre_ref, nrm_ref, msk_ref, mop_ref, *, nfb, gpb):
    t = pl.program_id(0)
    p = pl.program_id(1)
    t_tile = pre_ref.shape[2]
    cur = jax.lax.rem(t, 2)
    prv = 1 - cur

    @pl.when(p < nfb)
    def _step():
        _norms(pre_ref, nrm_ref, jnp.where(p == 0, prv, cur),
               jnp.maximum(p - 1, 0), gpb)
        h = t_tile // 2
        for i in range(2):
            blk = jnp.dot(x_ref[i * h:(i + 1) * h, :], we_ref[...],
                          preferred_element_type=jnp.float32)
            pre_ref[cur, p, i * h:(i + 1) * h, :] = blk + be_ref[p]
        for i in range(2):
            sl = slice(i * h, (i + 1) * h)
            acc = jnp.dot(mop_ref[jax.lax.rem(p, 2), sl, :], wd_ref[...],
                          preferred_element_type=jnp.float32)
            base = jnp.where(p == 0, bd_ref[...], out_ref[sl, :])
            out_ref[sl, :] = base + acc
        _prep(pre_ref, msk_ref, mop_ref, prv, jnp.minimum(p + 1, nfb - 1),
              gpb)

    @pl.when(p == nfb)
    def _select():
        _norms(pre_ref, nrm_ref, cur, nfb - 1, gpb)
        bits = jax.lax.bitcast_convert_type(nrm_ref[...], jnp.int32)

        def body(_, carry):
            lo_b, hi_b = carry
            mid = lo_b + ((hi_b - lo_b) >> 1)
            cnt = jnp.sum((bits >= mid).astype(jnp.int32), axis=0,
                          keepdims=True)
            ok = cnt >= _K_GROUPS
            return jnp.where(ok, mid, lo_b), jnp.where(ok, hi_b, mid)

        lo0 = jnp.zeros((1, t_tile), jnp.int32)
        hi0 = jnp.full((1, t_tile), jnp.int32(0x7F800000))
        thr, _ = jax.lax.fori_loop(0, 31, body, (lo0, hi0))
        msk_ref[...] = (bits >= thr).astype(jnp.bfloat16)
        _prep(pre_ref, msk_ref, mop_ref, cur, 0, gpb)


def kernel(x, W_enc, W_dec, b_enc, b_dec):
    tokens, d_model = x.shape
    d_sae = W_enc.shape[1]
    nfb = d_sae // _FB
    gpb = _FB // _GROUP_RANK
    n_tiles = tokens // _T

    x16 = x.astype(jnp.bfloat16)
    we16 = W_enc.astype(jnp.bfloat16)
    wd16 = W_dec.astype(jnp.bfloat16)
    be3 = b_enc.reshape(nfb, 1, _FB)
    bd2 = b_dec.reshape(1, d_model)

    grid = (n_tiles + 1, nfb + 1)
    body = functools.partial(_fused, nfb=nfb, gpb=gpb)
    return pl.pallas_call(
        body,
        grid=grid,
        in_specs=[
            pl.BlockSpec((_T, d_model),
                         lambda t, p, m=n_tiles: (jnp.minimum(t, m - 1), 0)),
            pl.BlockSpec((d_model, _FB),
                         lambda t, p, n=nfb: (0, jnp.minimum(p, n - 1))),
            pl.BlockSpec((_FB, d_model),
                         lambda t, p, n=nfb: (jnp.minimum(p, n - 1), 0)),
            pl.BlockSpec((nfb, 1, _FB), lambda t, p: (0, 0, 0)),
            pl.BlockSpec((1, d_model), lambda t, p: (0, 0)),
        ],
        out_specs=pl.BlockSpec((_T, d_model),
                               lambda t, p: (jnp.maximum(t - 1, 0), 0)),
        out_shape=jax.ShapeDtypeStruct((tokens, d_model), jnp.float32),
        scratch_shapes=[
            pltpu.VMEM((2, nfb, _T, _FB), jnp.float32),
            pltpu.VMEM((nfb * gpb, _T), jnp.float32),
            pltpu.VMEM((nfb * gpb, _T), jnp.bfloat16),
            pltpu.VMEM((2, _T, _FB), jnp.bfloat16),
        ],
        compiler_params=pltpu.CompilerParams(
            dimension_semantics=("arbitrary", "arbitrary"),
            vmem_limit_bytes=64 * 1024 * 1024,
        ),
    )(x16, we16, wd16, be3, bd2)
